# baseline (speedup 1.0000x reference)
import jax
import jax.numpy as jnp
from jax.experimental import pallas as pl
from jax.experimental.pallas import tpu as pltpu

HIDDEN = 2048
EPS = 1e-6
ROUTING_WEIGHT = 0.1
TM = 512


def _fused_kernel(x_ref, wr_ref, gp_ref, gs_ref, wp_ref, out_ref, part_ref):
    xb = x_ref[...]
    var = jnp.mean(xb * xb, axis=1, keepdims=True)
    xn = xb * jax.lax.rsqrt(var + EPS)

    wr = wr_ref[...]
    l0 = jnp.sum(xb * wr[0:1, :], axis=1, keepdims=True)
    l1 = jnp.sum(xb * wr[1:2, :], axis=1, keepdims=True)
    m = jnp.maximum(l0, l1)
    e0 = jnp.exp(l0 - m)
    e1 = jnp.exp(l1 - m)
    denom = e0 + e1
    p = e0 / denom
    s = e1 / denom

    a = xn * gp_ref[...]
    proc = jnp.dot(a, wp_ref[...], preferred_element_type=jnp.float32)
    out_ref[...] = proc + (s * gs_ref[...]) * xn

    p_sum = jnp.sum(p)
    s_sum = jnp.sum(s)
    row = jax.lax.broadcasted_iota(jnp.int32, (8, 128), 0)
    col = jax.lax.broadcasted_iota(jnp.int32, (8, 128), 1)
    tile = jnp.where((row == 0) & (col == 0), p_sum, 0.0) + jnp.where(
        (row == 0) & (col == 1), s_sum, 0.0
    )
    part_ref[...] = tile[None].astype(jnp.float32)


def kernel(x, W_router, g_process, g_skip, W_proc):
    batch, seq, hidden = x.shape
    rows = batch * seq
    m_tiles = rows // TM
    x2 = x.reshape(rows, hidden)

    out, partials = pl.pallas_call(
        _fused_kernel,
        grid=(m_tiles,),
        in_specs=[
            pl.BlockSpec((TM, hidden), lambda i: (i, 0)),
            pl.BlockSpec((2, hidden), lambda i: (0, 0)),
            pl.BlockSpec((1, hidden), lambda i: (0, 0)),
            pl.BlockSpec((1, hidden), lambda i: (0, 0)),
            pl.BlockSpec((hidden, hidden), lambda i: (0, 0)),
        ],
        out_specs=[
            pl.BlockSpec((TM, hidden), lambda i: (i, 0)),
            pl.BlockSpec((1, 8, 128), lambda i: (i, 0, 0)),
        ],
        out_shape=[
            jax.ShapeDtypeStruct((rows, hidden), jnp.float32),
            jax.ShapeDtypeStruct((m_tiles, 8, 128), jnp.float32),
        ],
        compiler_params=pltpu.CompilerParams(
            dimension_semantics=("arbitrary",),
        ),
    )(
        x2,
        W_router,
        g_process.reshape(1, hidden),
        g_skip.reshape(1, hidden),
        W_proc.astype(jnp.bfloat16),
    )

    output = out.reshape(batch, seq, hidden)

    tiles_per_batch = m_tiles // batch
    p_sums = partials[:, 0, 0].reshape(batch, tiles_per_batch).sum(axis=1)
    s_sums = partials[:, 0, 1].reshape(batch, tiles_per_batch).sum(axis=1)
    p_mean = p_sums / seq
    s_mean = s_sums / seq
    balance = jnp.var(p_mean, ddof=1) + jnp.var(s_mean, ddof=1)
    routing_loss = balance * ROUTING_WEIGHT
    return (output, routing_loss)


# speedup vs baseline: 1.0011x; 1.0011x over previous
import jax
import jax.numpy as jnp
from jax.experimental import pallas as pl
from jax.experimental.pallas import tpu as pltpu

HIDDEN = 2048
EPS = 1e-6
ROUTING_WEIGHT = 0.1
TM = 512


def _fused_kernel(x_ref, wr_ref, gp_ref, gs_ref, wp_ref, out_ref, part_ref):
    xb = x_ref[...]
    var = jnp.mean(xb * xb, axis=1, keepdims=True)
    xn = xb * jax.lax.rsqrt(var + EPS)

    wr = wr_ref[...]
    l0 = jnp.sum(xb * wr[0:1, :], axis=1, keepdims=True)
    l1 = jnp.sum(xb * wr[1:2, :], axis=1, keepdims=True)
    m = jnp.maximum(l0, l1)
    e0 = jnp.exp(l0 - m)
    e1 = jnp.exp(l1 - m)
    denom = e0 + e1
    p = e0 / denom
    s = e1 / denom

    a = (xn * gp_ref[...]).astype(jnp.bfloat16)
    proc = jnp.dot(a, wp_ref[...], preferred_element_type=jnp.float32)
    out_ref[...] = proc + (s * gs_ref[...]) * xn

    p_sum = jnp.sum(p)
    s_sum = jnp.sum(s)
    row = jax.lax.broadcasted_iota(jnp.int32, (8, 128), 0)
    col = jax.lax.broadcasted_iota(jnp.int32, (8, 128), 1)
    tile = jnp.where((row == 0) & (col == 0), p_sum, 0.0) + jnp.where(
        (row == 0) & (col == 1), s_sum, 0.0
    )
    part_ref[...] = tile[None].astype(jnp.float32)


def kernel(x, W_router, g_process, g_skip, W_proc):
    batch, seq, hidden = x.shape
    rows = batch * seq
    m_tiles = rows // TM
    x2 = x.reshape(rows, hidden)

    out, partials = pl.pallas_call(
        _fused_kernel,
        grid=(m_tiles,),
        in_specs=[
            pl.BlockSpec((TM, hidden), lambda i: (i, 0)),
            pl.BlockSpec((2, hidden), lambda i: (0, 0)),
            pl.BlockSpec((1, hidden), lambda i: (0, 0)),
            pl.BlockSpec((1, hidden), lambda i: (0, 0)),
            pl.BlockSpec((hidden, hidden), lambda i: (0, 0)),
        ],
        out_specs=[
            pl.BlockSpec((TM, hidden), lambda i: (i, 0)),
            pl.BlockSpec((1, 8, 128), lambda i: (i, 0, 0)),
        ],
        out_shape=[
            jax.ShapeDtypeStruct((rows, hidden), jnp.float32),
            jax.ShapeDtypeStruct((m_tiles, 8, 128), jnp.float32),
        ],
        compiler_params=pltpu.CompilerParams(
            dimension_semantics=("arbitrary",),
        ),
    )(
        x2,
        W_router,
        g_process.reshape(1, hidden),
        g_skip.reshape(1, hidden),
        W_proc.astype(jnp.bfloat16),
    )

    output = out.reshape(batch, seq, hidden)

    tiles_per_batch = m_tiles // batch
    p_sums = partials[:, 0, 0].reshape(batch, tiles_per_batch).sum(axis=1)
    s_sums = partials[:, 0, 1].reshape(batch, tiles_per_batch).sum(axis=1)
    p_mean = p_sums / seq
    s_mean = s_sums / seq
    balance = jnp.var(p_mean, ddof=1) + jnp.var(s_mean, ddof=1)
    routing_loss = balance * ROUTING_WEIGHT
    return (output, routing_loss)


# speedup vs baseline: 1.0696x; 1.0684x over previous
import jax
import jax.numpy as jnp
from jax.experimental import pallas as pl
from jax.experimental.pallas import tpu as pltpu

HIDDEN = 2048
EPS = 1e-6
ROUTING_WEIGHT = 0.1
TM = 512


def _fused_kernel(x_ref, wr_ref, gp_ref, gs_ref, wp_ref, out_ref, part_ref):
    xb = x_ref[...]
    a = xb * gp_ref[...]
    proc = jnp.dot(a, wp_ref[...], preferred_element_type=jnp.float32)

    var = jnp.mean(xb * xb, axis=1, keepdims=True)
    r = jax.lax.rsqrt(var + EPS)

    wr = wr_ref[...]
    l0 = jnp.sum(xb * wr[0:1, :], axis=1, keepdims=True)
    l1 = jnp.sum(xb * wr[1:2, :], axis=1, keepdims=True)
    m = jnp.maximum(l0, l1)
    e0 = jnp.exp(l0 - m)
    e1 = jnp.exp(l1 - m)
    denom = e0 + e1
    p = e0 / denom
    s = e1 / denom

    out_ref[...] = r * proc + (s * r) * (gs_ref[...] * xb)

    p_sum = jnp.sum(p)
    s_sum = jnp.sum(s)
    row = jax.lax.broadcasted_iota(jnp.int32, (8, 128), 0)
    col = jax.lax.broadcasted_iota(jnp.int32, (8, 128), 1)
    tile = jnp.where((row == 0) & (col == 0), p_sum, 0.0) + jnp.where(
        (row == 0) & (col == 1), s_sum, 0.0
    )
    part_ref[...] = tile[None].astype(jnp.float32)


def kernel(x, W_router, g_process, g_skip, W_proc):
    batch, seq, hidden = x.shape
    rows = batch * seq
    m_tiles = rows // TM
    x2 = x.reshape(rows, hidden)

    out, partials = pl.pallas_call(
        _fused_kernel,
        grid=(m_tiles,),
        in_specs=[
            pl.BlockSpec((TM, hidden), lambda i: (i, 0)),
            pl.BlockSpec((2, hidden), lambda i: (0, 0)),
            pl.BlockSpec((1, hidden), lambda i: (0, 0)),
            pl.BlockSpec((1, hidden), lambda i: (0, 0)),
            pl.BlockSpec((hidden, hidden), lambda i: (0, 0)),
        ],
        out_specs=[
            pl.BlockSpec((TM, hidden), lambda i: (i, 0)),
            pl.BlockSpec((1, 8, 128), lambda i: (i, 0, 0)),
        ],
        out_shape=[
            jax.ShapeDtypeStruct((rows, hidden), jnp.float32),
            jax.ShapeDtypeStruct((m_tiles, 8, 128), jnp.float32),
        ],
        compiler_params=pltpu.CompilerParams(
            dimension_semantics=("arbitrary",),
        ),
    )(
        x2,
        W_router,
        g_process.reshape(1, hidden),
        g_skip.reshape(1, hidden),
        W_proc,
    )

    output = out.reshape(batch, seq, hidden)

    tiles_per_batch = m_tiles // batch
    p_sums = partials[:, 0, 0].reshape(batch, tiles_per_batch).sum(axis=1)
    s_sums = partials[:, 0, 1].reshape(batch, tiles_per_batch).sum(axis=1)
    p_mean = p_sums / seq
    s_mean = s_sums / seq
    balance = jnp.var(p_mean, ddof=1) + jnp.var(s_mean, ddof=1)
    routing_loss = balance * ROUTING_WEIGHT
    return (output, routing_loss)


# speedup vs baseline: 1.1179x; 1.0452x over previous
import jax
import jax.numpy as jnp
from jax.experimental import pallas as pl
from jax.experimental.pallas import tpu as pltpu

HIDDEN = 2048
EPS = 1e-6
ROUTING_WEIGHT = 0.1
TM = 512


def _fused_kernel(x_ref, wr_ref, gp_ref, gs_ref, wp_ref, out_ref, part_ref):
    xb = x_ref[...]
    var = jnp.mean(xb * xb, axis=1, keepdims=True)
    xn = xb * jax.lax.rsqrt(var + EPS)

    wr = wr_ref[...]
    l0 = jnp.sum(xb * wr[0:1, :], axis=1, keepdims=True)
    l1 = jnp.sum(xb * wr[1:2, :], axis=1, keepdims=True)
    m = jnp.maximum(l0, l1)
    e0 = jnp.exp(l0 - m)
    e1 = jnp.exp(l1 - m)
    denom = e0 + e1
    p = e0 / denom
    s = e1 / denom

    a = xn * gp_ref[...]
    proc = jnp.dot(a, wp_ref[...], preferred_element_type=jnp.float32)
    out_ref[...] = proc + (s * gs_ref[...]) * xn

    p_sum = jnp.sum(p)
    s_sum = jnp.sum(s)
    row = jax.lax.broadcasted_iota(jnp.int32, (8, 128), 0)
    col = jax.lax.broadcasted_iota(jnp.int32, (8, 128), 1)
    tile = jnp.where((row == 0) & (col == 0), p_sum, 0.0) + jnp.where(
        (row == 0) & (col == 1), s_sum, 0.0
    )
    part_ref[...] = tile[None].astype(jnp.float32)


def kernel(x, W_router, g_process, g_skip, W_proc):
    batch, seq, hidden = x.shape
    rows = batch * seq
    m_tiles = rows // TM
    x2 = x.reshape(rows, hidden)

    out, partials = pl.pallas_call(
        _fused_kernel,
        grid=(m_tiles,),
        in_specs=[
            pl.BlockSpec((TM, hidden), lambda i: (i, 0)),
            pl.BlockSpec((2, hidden), lambda i: (0, 0)),
            pl.BlockSpec((1, hidden), lambda i: (0, 0)),
            pl.BlockSpec((1, hidden), lambda i: (0, 0)),
            pl.BlockSpec((hidden, hidden), lambda i: (0, 0)),
        ],
        out_specs=[
            pl.BlockSpec((TM, hidden), lambda i: (i, 0)),
            pl.BlockSpec((1, 8, 128), lambda i: (i, 0, 0)),
        ],
        out_shape=[
            jax.ShapeDtypeStruct((rows, hidden), jnp.float32),
            jax.ShapeDtypeStruct((m_tiles, 8, 128), jnp.float32),
        ],
        compiler_params=pltpu.CompilerParams(
            dimension_semantics=("arbitrary",),
        ),
    )(
        x2,
        W_router,
        g_process.reshape(1, hidden),
        g_skip.reshape(1, hidden),
        W_proc,
    )

    output = out.reshape(batch, seq, hidden)

    tiles_per_batch = m_tiles // batch
    p_sums = partials[:, 0, 0].reshape(batch, tiles_per_batch).sum(axis=1)
    s_sums = partials[:, 0, 1].reshape(batch, tiles_per_batch).sum(axis=1)
    p_mean = p_sums / seq
    s_mean = s_sums / seq
    balance = jnp.var(p_mean, ddof=1) + jnp.var(s_mean, ddof=1)
    routing_loss = balance * ROUTING_WEIGHT
    return (output, routing_loss)
